# baseline (speedup 1.0000x reference)
import functools
import math

import jax
import jax.numpy as jnp
from jax import lax
from jax.experimental import pallas as pl
from jax.experimental.pallas import tpu as pltpu
from jax.experimental.pallas import tpu_sc as plsc

NN = 10000
EE = 320000
H = 8
NSUB = 16
NCORE = 2
B = 80
EC = EE // NSUB
NB = EC // B
G = B // 16
RPS = 624
CHUNKS = [(i * B, B) for i in range(RPS // B)] + [(RPS - RPS % B, RPS % B)]
W = 128
HPC = 4
DIMS_L = [(128, 256), (256, 256), (256, 256), (256, 128)]


def _make_proj(din, hc):
    D = hc // 2
    bn = 400
    grid = NN // bn
    wdim = 4 * hc

    def body(x_ref, w_ref, b_ref, q_ref, kv_ref, s_ref):
        acc = jnp.dot(x_ref[...], w_ref[...],
                      preferred_element_type=jnp.float32) + b_ref[...]
        pad = jnp.zeros((bn, W - D), jnp.float32) if D < W else None
        q = acc[:, 0:hc]
        k = acc[:, hc:2 * hc]
        v = acc[:, 2 * hc:3 * hc]
        for half in range(2):
            qh = q[:, half * D:(half + 1) * D]
            kh = k[:, half * D:(half + 1) * D]
            vh = v[:, half * D:(half + 1) * D]
            if pad is not None:
                qh = jnp.concatenate([qh, pad], axis=1)
                kh = jnp.concatenate([kh, pad], axis=1)
                vh = jnp.concatenate([vh, pad], axis=1)
            q_ref[half] = qh
            kv_ref[half] = jnp.concatenate([kh, vh], axis=1)
        s_ref[...] = acc[:, 3 * hc:]

    return pl.pallas_call(
        body,
        grid=(grid,),
        in_specs=[
            pl.BlockSpec((bn, din), lambda i: (i, 0)),
            pl.BlockSpec((din, wdim), lambda i: (0, 0)),
            pl.BlockSpec((1, wdim), lambda i: (0, 0)),
        ],
        out_specs=[
            pl.BlockSpec((2, bn, W), lambda i: (0, i, 0)),
            pl.BlockSpec((2, bn, 2 * W), lambda i: (0, i, 0)),
            pl.BlockSpec((bn, hc), lambda i: (i, 0)),
        ],
        out_shape=[
            jax.ShapeDtypeStruct((2, NN, W), jnp.float32),
            jax.ShapeDtypeStruct((2, NN, 2 * W), jnp.float32),
            jax.ShapeDtypeStruct((NN, hc), jnp.float32),
        ],
    )


def _make_edge(hc):
    CH = hc // 2
    c = hc // H
    inv = 1.0 / math.sqrt(c)
    mesh = plsc.VectorSubcoreMesh(core_axis_name="c", subcore_axis_name="s")

    @functools.partial(
        pl.kernel,
        mesh=mesh,
        compiler_params=pltpu.CompilerParams(needs_layout_passes=False),
        out_type=(
            jax.ShapeDtypeStruct((NCORE * NN, W), jnp.float32),
            jax.ShapeDtypeStruct((NCORE * NN,), jnp.float32),
            jax.ShapeDtypeStruct((NCORE * NN,), jnp.float32),
            jax.ShapeDtypeStruct((NCORE * NN,), jnp.float32),
            jax.ShapeDtypeStruct((NCORE * NN,), jnp.float32),
        ),
        scratch_types=[
            pltpu.VMEM((B, W), jnp.float32),
            pltpu.VMEM((B, 2 * W), jnp.float32),
            pltpu.VMEM((HPC, B), jnp.float32),
            pltpu.VMEM((B,), jnp.int32),
            pltpu.VMEM((B,), jnp.int32),
            pltpu.VMEM((B,), jnp.int32),
            pltpu.VMEM_SHARED((NN, W), jnp.float32),
            pltpu.VMEM_SHARED((NN,), jnp.float32),
            pltpu.VMEM_SHARED((NN,), jnp.float32),
            pltpu.VMEM_SHARED((NN,), jnp.float32),
            pltpu.VMEM_SHARED((NN,), jnp.float32),
            pltpu.SemaphoreType.DMA,
        ],
    )
    def edge_kernel(q_hbm, kv_hbm, src_hbm, dst_hbm,
                    num_out, den_out0, den_out1, den_out2, den_out3,
                    qb, kvb, denT, dstv, dstg, srcg,
                    num_sp, den_sp0, den_sp1, den_sp2, den_sp3, sem):
        den_sps = [den_sp0, den_sp1, den_sp2, den_sp3]
        den_outs = [den_out0, den_out1, den_out2, den_out3]
        core = lax.axis_index("c")
        sub = lax.axis_index("s")
        lane = lax.iota(jnp.int32, 16)
        zero16 = jnp.zeros((16,), jnp.float32)

        def zrow(r, carry):
            for j in range(W // 16):
                qb[r, pl.ds(j * 16, 16)] = zero16
            return carry
        lax.fori_loop(0, B, zrow, 0)
        for h in range(HPC):
            for j in range(B // 16):
                denT[h, pl.ds(j * 16, 16)] = zero16

        rbase = sub * RPS
        for start, rows in CHUNKS:
            pltpu.sync_copy(qb.at[pl.ds(0, rows)],
                            num_sp.at[pl.ds(rbase + start, rows)])
            for h in range(HPC):
                pltpu.sync_copy(denT.at[h].at[pl.ds(0, rows)],
                                den_sps[h].at[pl.ds(rbase + start, rows)])

        @pl.when(sub == NSUB - 1)
        def _init_tail():
            pltpu.sync_copy(qb.at[pl.ds(0, 16)],
                            num_sp.at[pl.ds(NSUB * RPS, 16)])
            for h in range(HPC):
                pltpu.sync_copy(denT.at[h].at[pl.ds(0, 16)],
                                den_sps[h].at[pl.ds(NSUB * RPS, 16)])

        plsc.subcore_barrier()

        coff = core * NN
        ebase = sub * EC

        def block_body(b, carry):
            off = ebase + b * B
            pltpu.sync_copy(src_hbm.at[pl.ds(off, B)], srcg)
            pltpu.sync_copy(dst_hbm.at[pl.ds(off, B)], dstv)
            for i in range(B // 16):
                sl = pl.ds(i * 16, 16)
                srcg[sl] = srcg[sl] + coff
                dstg[sl] = dstv[sl] + coff
            cq = pltpu.async_copy(q_hbm.at[dstg], qb, sem)
            ckv = pltpu.async_copy(kv_hbm.at[srcg], kvb, sem)
            cq.wait()
            ckv.wait()

            def group_body(g, gcarry):
                row = g * 16 + lane
                exs = []
                for h in range(HPC):
                    acc = zero16
                    for cc in range(c):
                        colv = jnp.full((16,), h * c + cc, jnp.int32)
                        qv = plsc.load_gather(qb, [row, colv])
                        kv = plsc.load_gather(kvb, [row, colv])
                        acc = acc + qv * kv
                    ex = jnp.exp(acc * inv)
                    exs.append(ex)
                    denT[h, pl.ds(g * 16, 16)] = ex
                for ch in range(CH):
                    mv = plsc.load_gather(
                        kvb, [row, jnp.full((16,), W + ch, jnp.int32)])
                    plsc.store_scatter(
                        qb, [row, jnp.full((16,), ch, jnp.int32)],
                        mv * exs[ch // c])
                return gcarry
            lax.fori_loop(0, G, group_body, 0)

            pltpu.sync_copy(qb, num_sp.at[dstv], add=True)
            for h in range(HPC):
                pltpu.sync_copy(denT.at[h], den_sps[h].at[dstv], add=True)
            return carry
        lax.fori_loop(0, NB, block_body, 0)

        plsc.subcore_barrier()

        obase = coff + rbase
        for start, rows in CHUNKS:
            pltpu.sync_copy(num_sp.at[pl.ds(rbase + start, rows)],
                            qb.at[pl.ds(0, rows)])
            pltpu.sync_copy(qb.at[pl.ds(0, rows)],
                            num_out.at[pl.ds(obase + start, rows)])
            for h in range(HPC):
                pltpu.sync_copy(den_sps[h].at[pl.ds(rbase + start, rows)],
                                denT.at[h].at[pl.ds(0, rows)])
                pltpu.sync_copy(denT.at[h].at[pl.ds(0, rows)],
                                den_outs[h].at[pl.ds(obase + start, rows)])

        @pl.when(sub == NSUB - 1)
        def _out_tail():
            pltpu.sync_copy(num_sp.at[pl.ds(NSUB * RPS, 16)],
                            qb.at[pl.ds(0, 16)])
            pltpu.sync_copy(qb.at[pl.ds(0, 16)],
                            num_out.at[pl.ds(coff + NSUB * RPS, 16)])
            for h in range(HPC):
                pltpu.sync_copy(den_sps[h].at[pl.ds(NSUB * RPS, 16)],
                                denT.at[h].at[pl.ds(0, 16)])
                pltpu.sync_copy(denT.at[h].at[pl.ds(0, 16)],
                                den_outs[h].at[pl.ds(coff + NSUB * RPS, 16)])

    return edge_kernel


def _make_final(hc, use_gelu):
    D = hc // 2
    c = hc // H
    bn = 400
    grid = NN // bn

    def body(num_ref, den_ref, skip_ref, out_ref):
        ih = lax.broadcasted_iota(jnp.int32, (HPC, D), 0)
        ic = lax.broadcasted_iota(jnp.int32, (HPC, D), 1)
        R = (ic // c == ih).astype(jnp.float32)
        halves = []
        for half in range(2):
            dexp = jnp.dot(den_ref[half], R,
                           preferred_element_type=jnp.float32)
            halves.append(num_ref[half, :, :D] / (dexp + 1e-16))
        out = jnp.concatenate(halves, axis=1) + skip_ref[...]
        if use_gelu:
            out = 0.5 * out * (1.0 + lax.erf(out * (1.0 / math.sqrt(2.0))))
        out_ref[...] = out

    return pl.pallas_call(
        body,
        grid=(grid,),
        in_specs=[
            pl.BlockSpec((2, bn, W), lambda i: (0, i, 0)),
            pl.BlockSpec((2, bn, HPC), lambda i: (0, i, 0)),
            pl.BlockSpec((bn, hc), lambda i: (i, 0)),
        ],
        out_specs=pl.BlockSpec((bn, hc), lambda i: (i, 0)),
        out_shape=jax.ShapeDtypeStruct((NN, hc), jnp.float32),
    )


_PROJ = {}
_EDGE = {}
_FINAL = {}
for _l, (_din, _hc) in enumerate(DIMS_L):
    if (_din, _hc) not in _PROJ:
        _PROJ[(_din, _hc)] = _make_proj(_din, _hc)
    if _hc not in _EDGE:
        _EDGE[_hc] = _make_edge(_hc)
    if (_hc, _l < 3) not in _FINAL:
        _FINAL[(_hc, _l < 3)] = _make_final(_hc, _l < 3)


def kernel(x, edge_index,
           Wq0, bq0, Wk0, bk0, Wv0, bv0, Ws0, bs0,
           Wq1, bq1, Wk1, bk1, Wv1, bv1, Ws1, bs1,
           Wq2, bq2, Wk2, bk2, Wv2, bv2, Ws2, bs2,
           Wq3, bq3, Wk3, bk3, Wv3, bv3, Ws3, bs3):
    params = (Wq0, bq0, Wk0, bk0, Wv0, bv0, Ws0, bs0,
              Wq1, bq1, Wk1, bk1, Wv1, bv1, Ws1, bs1,
              Wq2, bq2, Wk2, bk2, Wv2, bv2, Ws2, bs2,
              Wq3, bq3, Wk3, bk3, Wv3, bv3, Ws3, bs3)
    srcp = edge_index[0]
    dstp = edge_index[1]
    h = x
    for l, (din, hc) in enumerate(DIMS_L):
        Wq, bq, Wk, bk, Wv, bv, Ws, bs = params[8 * l:8 * (l + 1)]
        Wc = jnp.concatenate([Wq, Wk, Wv, Ws], axis=1)
        bc = jnp.concatenate([bq, bk, bv, bs]).reshape(1, -1)
        q3, kv3, skip = _PROJ[(din, hc)](h, Wc, bc)
        num, d0, d1, d2, d3 = _EDGE[hc](q3.reshape(2 * NN, W),
                                        kv3.reshape(2 * NN, 2 * W),
                                        srcp, dstp)
        den = jnp.stack([d0, d1, d2, d3], axis=-1).reshape(2, NN, HPC)
        h = _FINAL[(hc, l < 3)](num.reshape(2, NN, W), den, skip)
    return h


# speedup vs baseline: 3.0740x; 3.0740x over previous
import functools
import math

import jax
import jax.numpy as jnp
from jax import lax
from jax.experimental import pallas as pl
from jax.experimental.pallas import tpu as pltpu
from jax.experimental.pallas import tpu_sc as plsc

NN = 10000
EE = 320000
H = 8
NSUB = 16
NCORE = 2
B = 80
EC = EE // NSUB
NB = EC // B
G = B // 16
RPS = 624
CHUNKS = [(i * B, B) for i in range(RPS // B)] + [(RPS - RPS % B, RPS % B)]
W = 128
HPC = 4
DIMS_L = [(128, 256), (256, 256), (256, 256), (256, 128)]


def _make_proj(din, hc):
    D = hc // 2
    bn = 400
    grid = NN // bn
    wdim = 4 * hc

    def body(x_ref, w_ref, b_ref, q_ref, kv_ref, s_ref):
        acc = jnp.dot(x_ref[...], w_ref[...],
                      preferred_element_type=jnp.float32) + b_ref[...]
        pad = jnp.zeros((bn, W - D), jnp.float32) if D < W else None
        q = acc[:, 0:hc]
        k = acc[:, hc:2 * hc]
        v = acc[:, 2 * hc:3 * hc]
        for half in range(2):
            qh = q[:, half * D:(half + 1) * D]
            kh = k[:, half * D:(half + 1) * D]
            vh = v[:, half * D:(half + 1) * D]
            if pad is not None:
                qh = jnp.concatenate([qh, pad], axis=1)
                kh = jnp.concatenate([kh, pad], axis=1)
                vh = jnp.concatenate([vh, pad], axis=1)
            q_ref[half] = qh
            kv_ref[half] = jnp.concatenate([kh, vh], axis=1)
        s_ref[...] = acc[:, 3 * hc:]

    return pl.pallas_call(
        body,
        grid=(grid,),
        in_specs=[
            pl.BlockSpec((bn, din), lambda i: (i, 0)),
            pl.BlockSpec((din, wdim), lambda i: (0, 0)),
            pl.BlockSpec((1, wdim), lambda i: (0, 0)),
        ],
        out_specs=[
            pl.BlockSpec((2, bn, W), lambda i: (0, i, 0)),
            pl.BlockSpec((2, bn, 2 * W), lambda i: (0, i, 0)),
            pl.BlockSpec((bn, hc), lambda i: (i, 0)),
        ],
        out_shape=[
            jax.ShapeDtypeStruct((2, NN, W), jnp.float32),
            jax.ShapeDtypeStruct((2, NN, 2 * W), jnp.float32),
            jax.ShapeDtypeStruct((NN, hc), jnp.float32),
        ],
    )


def _make_edge(hc):
    CH = hc // 2
    c = hc // H
    inv = 1.0 / math.sqrt(c)
    mesh = plsc.VectorSubcoreMesh(core_axis_name="c", subcore_axis_name="s")

    @functools.partial(
        pl.kernel,
        mesh=mesh,
        compiler_params=pltpu.CompilerParams(needs_layout_passes=False),
        out_type=(
            jax.ShapeDtypeStruct((NCORE * NN, W), jnp.float32),
            jax.ShapeDtypeStruct((NCORE * NN,), jnp.float32),
            jax.ShapeDtypeStruct((NCORE * NN,), jnp.float32),
            jax.ShapeDtypeStruct((NCORE * NN,), jnp.float32),
            jax.ShapeDtypeStruct((NCORE * NN,), jnp.float32),
        ),
        scratch_types=[
            pltpu.VMEM((B, W), jnp.float32),
            pltpu.VMEM((B, 2 * W), jnp.float32),
            pltpu.VMEM((HPC, B), jnp.float32),
            pltpu.VMEM((B,), jnp.int32),
            pltpu.VMEM((B,), jnp.int32),
            pltpu.VMEM((B,), jnp.int32),
            pltpu.VMEM_SHARED((NN, W), jnp.float32),
            pltpu.VMEM_SHARED((NN,), jnp.float32),
            pltpu.VMEM_SHARED((NN,), jnp.float32),
            pltpu.VMEM_SHARED((NN,), jnp.float32),
            pltpu.VMEM_SHARED((NN,), jnp.float32),
            pltpu.SemaphoreType.DMA,
        ],
    )
    def edge_kernel(q_hbm, kv_hbm, src_hbm, dst_hbm,
                    num_out, den_out0, den_out1, den_out2, den_out3,
                    qb, kvb, denT, dstv, dstg, srcg,
                    num_sp, den_sp0, den_sp1, den_sp2, den_sp3, sem):
        den_sps = [den_sp0, den_sp1, den_sp2, den_sp3]
        den_outs = [den_out0, den_out1, den_out2, den_out3]
        core = lax.axis_index("c")
        sub = lax.axis_index("s")
        lane = lax.iota(jnp.int32, 16)
        zero16 = jnp.zeros((16,), jnp.float32)

        def zrow(r, carry):
            for j in range(W // 16):
                qb[r, pl.ds(j * 16, 16)] = zero16
            return carry
        lax.fori_loop(0, B, zrow, 0)
        for h in range(HPC):
            for j in range(B // 16):
                denT[h, pl.ds(j * 16, 16)] = zero16

        rbase = sub * RPS
        for start, rows in CHUNKS:
            pltpu.sync_copy(qb.at[pl.ds(0, rows)],
                            num_sp.at[pl.ds(rbase + start, rows)])
            for h in range(HPC):
                pltpu.sync_copy(denT.at[h].at[pl.ds(0, rows)],
                                den_sps[h].at[pl.ds(rbase + start, rows)])

        @pl.when(sub == NSUB - 1)
        def _init_tail():
            pltpu.sync_copy(qb.at[pl.ds(0, 16)],
                            num_sp.at[pl.ds(NSUB * RPS, 16)])
            for h in range(HPC):
                pltpu.sync_copy(denT.at[h].at[pl.ds(0, 16)],
                                den_sps[h].at[pl.ds(NSUB * RPS, 16)])

        plsc.subcore_barrier()

        coff = core * NN
        ebase = sub * EC

        def block_body(b, carry):
            off = ebase + b * B
            pltpu.sync_copy(src_hbm.at[pl.ds(off, B)], srcg)
            pltpu.sync_copy(dst_hbm.at[pl.ds(off, B)], dstv)
            for i in range(B // 16):
                sl = pl.ds(i * 16, 16)
                srcg[sl] = srcg[sl] + coff
                dstg[sl] = dstv[sl] + coff
            cq = pltpu.async_copy(q_hbm.at[dstg], qb, sem)
            ckv = pltpu.async_copy(kv_hbm.at[srcg], kvb, sem)
            cq.wait()
            ckv.wait()

            def group_body(g, gcarry):
                alph = [zero16] * HPC
                for j in range(16):
                    erow = g * 16 + j
                    for h in range(HPC):
                        s = zero16
                        for c2 in range(c // 16):
                            sl = pl.ds(h * c + c2 * 16, 16)
                            s = s + qb[erow, sl] * kvb[erow, sl]
                        a = jnp.sum(s) * inv
                        alph[h] = jnp.where(lane == j, a, alph[h])
                exvs = []
                for h in range(HPC):
                    ex = jnp.exp(alph[h])
                    exvs.append(ex)
                    denT[h, pl.ds(g * 16, 16)] = ex
                for j in range(16):
                    erow = g * 16 + j
                    for ch2 in range(CH // 16):
                        h = (ch2 * 16) // c
                        mv = kvb[erow, pl.ds(W + ch2 * 16, 16)]
                        qb[erow, pl.ds(ch2 * 16, 16)] = mv * exvs[h][j]
                return gcarry
            lax.fori_loop(0, G, group_body, 0)

            pltpu.sync_copy(qb, num_sp.at[dstv], add=True)
            for h in range(HPC):
                pltpu.sync_copy(denT.at[h], den_sps[h].at[dstv], add=True)
            return carry
        lax.fori_loop(0, NB, block_body, 0)

        plsc.subcore_barrier()

        obase = coff + rbase
        for start, rows in CHUNKS:
            pltpu.sync_copy(num_sp.at[pl.ds(rbase + start, rows)],
                            qb.at[pl.ds(0, rows)])
            pltpu.sync_copy(qb.at[pl.ds(0, rows)],
                            num_out.at[pl.ds(obase + start, rows)])
            for h in range(HPC):
                pltpu.sync_copy(den_sps[h].at[pl.ds(rbase + start, rows)],
                                denT.at[h].at[pl.ds(0, rows)])
                pltpu.sync_copy(denT.at[h].at[pl.ds(0, rows)],
                                den_outs[h].at[pl.ds(obase + start, rows)])

        @pl.when(sub == NSUB - 1)
        def _out_tail():
            pltpu.sync_copy(num_sp.at[pl.ds(NSUB * RPS, 16)],
                            qb.at[pl.ds(0, 16)])
            pltpu.sync_copy(qb.at[pl.ds(0, 16)],
                            num_out.at[pl.ds(coff + NSUB * RPS, 16)])
            for h in range(HPC):
                pltpu.sync_copy(den_sps[h].at[pl.ds(NSUB * RPS, 16)],
                                denT.at[h].at[pl.ds(0, 16)])
                pltpu.sync_copy(denT.at[h].at[pl.ds(0, 16)],
                                den_outs[h].at[pl.ds(coff + NSUB * RPS, 16)])

    return edge_kernel


def _make_final(hc, use_gelu):
    D = hc // 2
    c = hc // H
    bn = 400
    grid = NN // bn

    def body(num_ref, den_ref, skip_ref, out_ref):
        ih = lax.broadcasted_iota(jnp.int32, (HPC, D), 0)
        ic = lax.broadcasted_iota(jnp.int32, (HPC, D), 1)
        R = (ic // c == ih).astype(jnp.float32)
        halves = []
        for half in range(2):
            dexp = jnp.dot(den_ref[half], R,
                           preferred_element_type=jnp.float32)
            halves.append(num_ref[half, :, :D] / (dexp + 1e-16))
        out = jnp.concatenate(halves, axis=1) + skip_ref[...]
        if use_gelu:
            out = 0.5 * out * (1.0 + lax.erf(out * (1.0 / math.sqrt(2.0))))
        out_ref[...] = out

    return pl.pallas_call(
        body,
        grid=(grid,),
        in_specs=[
            pl.BlockSpec((2, bn, W), lambda i: (0, i, 0)),
            pl.BlockSpec((2, bn, HPC), lambda i: (0, i, 0)),
            pl.BlockSpec((bn, hc), lambda i: (i, 0)),
        ],
        out_specs=pl.BlockSpec((bn, hc), lambda i: (i, 0)),
        out_shape=jax.ShapeDtypeStruct((NN, hc), jnp.float32),
    )


_PROJ = {}
_EDGE = {}
_FINAL = {}
for _l, (_din, _hc) in enumerate(DIMS_L):
    if (_din, _hc) not in _PROJ:
        _PROJ[(_din, _hc)] = _make_proj(_din, _hc)
    if _hc not in _EDGE:
        _EDGE[_hc] = _make_edge(_hc)
    if (_hc, _l < 3) not in _FINAL:
        _FINAL[(_hc, _l < 3)] = _make_final(_hc, _l < 3)


def kernel(x, edge_index,
           Wq0, bq0, Wk0, bk0, Wv0, bv0, Ws0, bs0,
           Wq1, bq1, Wk1, bk1, Wv1, bv1, Ws1, bs1,
           Wq2, bq2, Wk2, bk2, Wv2, bv2, Ws2, bs2,
           Wq3, bq3, Wk3, bk3, Wv3, bv3, Ws3, bs3):
    params = (Wq0, bq0, Wk0, bk0, Wv0, bv0, Ws0, bs0,
              Wq1, bq1, Wk1, bk1, Wv1, bv1, Ws1, bs1,
              Wq2, bq2, Wk2, bk2, Wv2, bv2, Ws2, bs2,
              Wq3, bq3, Wk3, bk3, Wv3, bv3, Ws3, bs3)
    srcp = edge_index[0]
    dstp = edge_index[1]
    h = x
    for l, (din, hc) in enumerate(DIMS_L):
        Wq, bq, Wk, bk, Wv, bv, Ws, bs = params[8 * l:8 * (l + 1)]
        Wc = jnp.concatenate([Wq, Wk, Wv, Ws], axis=1)
        bc = jnp.concatenate([bq, bk, bv, bs]).reshape(1, -1)
        q3, kv3, skip = _PROJ[(din, hc)](h, Wc, bc)
        num, d0, d1, d2, d3 = _EDGE[hc](q3.reshape(2 * NN, W),
                                        kv3.reshape(2 * NN, 2 * W),
                                        srcp, dstp)
        den = jnp.stack([d0, d1, d2, d3], axis=-1).reshape(2, NN, HPC)
        h = _FINAL[(hc, l < 3)](num.reshape(2, NN, W), den, skip)
    return h


# speedup vs baseline: 6.3306x; 2.0594x over previous
import functools
import math

import jax
import jax.numpy as jnp
from jax import lax
from jax.experimental import pallas as pl
from jax.experimental.pallas import tpu as pltpu
from jax.experimental.pallas import tpu_sc as plsc

NN = 10000
EE = 320000
H = 8
NSUB = 16
NCORE = 2
B = 48
EC = EE // NSUB
NB = 418
EPAD = NB * B - EC
G = B // 16
RPS = 624
CHUNKS = [(i * B, B) for i in range(RPS // B)]
W = 128
HPC = 4
DIMS_L = [(128, 256), (256, 256), (256, 256), (256, 128)]


def _make_proj(din, hc):
    D = hc // 2
    bn = 400
    grid = NN // bn
    wdim = 4 * hc

    def body(x_ref, w_ref, b_ref, q_ref, k_ref, v_ref, s_ref):
        acc = jnp.dot(x_ref[...], w_ref[...],
                      preferred_element_type=jnp.float32) + b_ref[...]
        pad = jnp.zeros((bn, W - D), jnp.float32) if D < W else None
        for t, ref in enumerate([q_ref, k_ref, v_ref]):
            blk = acc[:, t * hc:(t + 1) * hc]
            for half in range(2):
                hb = blk[:, half * D:(half + 1) * D]
                if pad is not None:
                    hb = jnp.concatenate([hb, pad], axis=1)
                ref[half] = hb
        s_ref[...] = acc[:, 3 * hc:]

    return pl.pallas_call(
        body,
        grid=(grid,),
        in_specs=[
            pl.BlockSpec((bn, din), lambda i: (i, 0)),
            pl.BlockSpec((din, wdim), lambda i: (0, 0)),
            pl.BlockSpec((1, wdim), lambda i: (0, 0)),
        ],
        out_specs=[
            pl.BlockSpec((2, bn, W), lambda i: (0, i, 0)),
            pl.BlockSpec((2, bn, W), lambda i: (0, i, 0)),
            pl.BlockSpec((2, bn, W), lambda i: (0, i, 0)),
            pl.BlockSpec((bn, hc), lambda i: (i, 0)),
        ],
        out_shape=[
            jax.ShapeDtypeStruct((2, NN, W), jnp.float32),
            jax.ShapeDtypeStruct((2, NN, W), jnp.float32),
            jax.ShapeDtypeStruct((2, NN, W), jnp.float32),
            jax.ShapeDtypeStruct((NN, hc), jnp.float32),
        ],
    )


def _make_edge(hc):
    CH = hc // 2
    c = hc // H
    inv = 1.0 / math.sqrt(c)
    mesh = plsc.VectorSubcoreMesh(core_axis_name="c", subcore_axis_name="s")

    @functools.partial(
        pl.kernel,
        mesh=mesh,
        compiler_params=pltpu.CompilerParams(needs_layout_passes=False),
        out_type=(
            jax.ShapeDtypeStruct((NCORE * NN, W), jnp.float32),
            jax.ShapeDtypeStruct((NCORE * NN,), jnp.float32),
            jax.ShapeDtypeStruct((NCORE * NN,), jnp.float32),
            jax.ShapeDtypeStruct((NCORE * NN,), jnp.float32),
            jax.ShapeDtypeStruct((NCORE * NN,), jnp.float32),
        ),
        scratch_types=[
            pltpu.VMEM((B, W), jnp.float32),
            pltpu.VMEM((B, W), jnp.float32),
            pltpu.VMEM((B, W), jnp.float32),
            pltpu.VMEM((B, W), jnp.float32),
            pltpu.VMEM((B, W), jnp.float32),
            pltpu.VMEM((HPC, B), jnp.float32),
            pltpu.VMEM((HPC, B), jnp.float32),
            pltpu.VMEM((B,), jnp.int32),
            pltpu.VMEM((B,), jnp.int32),
            pltpu.VMEM((B,), jnp.int32),
            pltpu.VMEM((B,), jnp.int32),
            pltpu.VMEM((B,), jnp.int32),
            pltpu.VMEM((B,), jnp.int32),
            pltpu.VMEM((B,), jnp.int32),
            pltpu.VMEM((B,), jnp.int32),
            pltpu.VMEM((B,), jnp.int32),
            pltpu.VMEM((B,), jnp.int32),
            pltpu.VMEM_SHARED((NN, W), jnp.float32),
            pltpu.VMEM_SHARED((NN,), jnp.float32),
            pltpu.VMEM_SHARED((NN,), jnp.float32),
            pltpu.VMEM_SHARED((NN,), jnp.float32),
            pltpu.VMEM_SHARED((NN,), jnp.float32),
            pltpu.SemaphoreType.DMA,
            pltpu.SemaphoreType.DMA,
            pltpu.SemaphoreType.DMA,
            pltpu.SemaphoreType.DMA,
            pltpu.SemaphoreType.DMA,
        ],
    )
    def edge_kernel(q_hbm, k_hbm, v_hbm, src_hbm, dst_hbm,
                    num_out, den_out0, den_out1, den_out2, den_out3,
                    qb0, qb1, kb0, kb1, vb, denT0, denT1,
                    sraw0, sraw1, draw0, draw1,
                    srcg0, srcg1, dstg0, dstg1, dstv0, dstv1,
                    num_sp, den_sp0, den_sp1, den_sp2, den_sp3,
                    gsem0, gsem1, vsem, ssem, isem):
        qb = [qb0, qb1]
        kb = [kb0, kb1]
        denT = [denT0, denT1]
        sraw = [sraw0, sraw1]
        draw = [draw0, draw1]
        srcg = [srcg0, srcg1]
        dstg = [dstg0, dstg1]
        dstv = [dstv0, dstv1]
        gsem = [gsem0, gsem1]
        den_sps = [den_sp0, den_sp1, den_sp2, den_sp3]
        den_outs = [den_out0, den_out1, den_out2, den_out3]
        core = lax.axis_index("c")
        sub = lax.axis_index("s")
        lane = lax.iota(jnp.int32, 16)
        zero16 = jnp.zeros((16,), jnp.float32)
        coff = core * NN
        ebase = sub * EC

        def zrow(r, carry):
            for j in range(W // 16):
                qb0[r, pl.ds(j * 16, 16)] = zero16
            return carry
        lax.fori_loop(0, B, zrow, 0)
        for h in range(HPC):
            for j in range(B // 16):
                denT0[h, pl.ds(j * 16, 16)] = zero16

        rbase = sub * RPS
        for start, rows in CHUNKS:
            pltpu.sync_copy(qb0.at[pl.ds(0, rows)],
                            num_sp.at[pl.ds(rbase + start, rows)])
            for h in range(HPC):
                pltpu.sync_copy(denT0.at[h].at[pl.ds(0, rows)],
                                den_sps[h].at[pl.ds(rbase + start, rows)])

        @pl.when(sub == NSUB - 1)
        def _init_tail():
            pltpu.sync_copy(qb0.at[pl.ds(0, 16)],
                            num_sp.at[pl.ds(NSUB * RPS, 16)])
            for h in range(HPC):
                pltpu.sync_copy(denT0.at[h].at[pl.ds(0, 16)],
                                den_sps[h].at[pl.ds(NSUB * RPS, 16)])

        plsc.subcore_barrier()

        def idx_load(t, p):
            off = ebase + t * B
            a = pltpu.async_copy(src_hbm.at[pl.ds(off, B)], sraw[p], isem)
            b_ = pltpu.async_copy(dst_hbm.at[pl.ds(off, B)], draw[p], isem)
            return a, b_

        def idx_wait(p):
            pltpu.make_async_copy(src_hbm.at[pl.ds(0, B)], sraw[p],
                                  isem).wait()
            pltpu.make_async_copy(dst_hbm.at[pl.ds(0, B)], draw[p],
                                  isem).wait()

        def offset_idx(p):
            for i in range(B // 16):
                sl = pl.ds(i * 16, 16)
                srcg[p][sl] = sraw[p][sl] + coff
                dstg[p][sl] = draw[p][sl] + coff
                dstv[p][sl] = draw[p][sl]

        def gather_issue(p):
            pltpu.async_copy(q_hbm.at[dstg[p]], qb[p], gsem[p])
            pltpu.async_copy(k_hbm.at[srcg[p]], kb[p], gsem[p])

        def gather_wait(p):
            pltpu.make_async_copy(q_hbm.at[dstg[p]], qb[p], gsem[p]).wait()
            pltpu.make_async_copy(k_hbm.at[srcg[p]], kb[p], gsem[p]).wait()

        def scatter_issue(p):
            pltpu.async_copy(qb[p], num_sp.at[dstv[p]], ssem, add=True)
            for h in range(HPC):
                pltpu.async_copy(denT[p].at[h], den_sps[h].at[dstv[p]],
                                 ssem, add=True)

        def scatter_drain(p):
            pltpu.make_async_copy(qb[p], num_sp.at[dstv[p]], ssem).wait()
            for h in range(HPC):
                pltpu.make_async_copy(denT[p].at[h],
                                      den_sps[h].at[dstv[p]], ssem).wait()

        a0, b0 = idx_load(0, 0)
        a0.wait()
        b0.wait()
        offset_idx(0)
        gather_issue(0)
        idx_load(1, 1)

        def pair_body(i, carry):
            for p in (0, 1):
                t = 2 * i + p

                @pl.when(t > 0)
                def _drain_prev():
                    scatter_drain(1 - p)

                gather_wait(p)
                pltpu.async_copy(v_hbm.at[srcg[p]], vb, vsem)

                @pl.when(t + 1 < NB)
                def _stage_next():
                    idx_wait(1 - p)
                    offset_idx(1 - p)
                    gather_issue(1 - p)

                @pl.when(t + 2 < NB)
                def _fetch_next_idx():
                    idx_load(t + 2, p)

                def alpha_body(g, gcarry):
                    alph = [zero16] * HPC
                    for j in range(16):
                        erow = g * 16 + j
                        for h in range(HPC):
                            s = zero16
                            for c2 in range(c // 16):
                                sl = pl.ds(h * c + c2 * 16, 16)
                                s = s + qb[p][erow, sl] * kb[p][erow, sl]
                            a = jnp.sum(s) * inv
                            alph[h] = jnp.where(lane == j, a, alph[h])
                    live = (t * B + g * 16 + lane) < EC
                    for h in range(HPC):
                        ex = jnp.where(live, jnp.exp(alph[h]), 0.0)
                        denT[p][h, pl.ds(g * 16, 16)] = ex
                    return gcarry
                lax.fori_loop(0, G, alpha_body, 0)

                pltpu.make_async_copy(v_hbm.at[srcg[p]], vb, vsem).wait()

                def msg_body(g, gcarry):
                    exvs = [denT[p][h, pl.ds(g * 16, 16)]
                            for h in range(HPC)]
                    for j in range(16):
                        erow = g * 16 + j
                        for ch2 in range(CH // 16):
                            h = (ch2 * 16) // c
                            mv = vb[erow, pl.ds(ch2 * 16, 16)]
                            qb[p][erow, pl.ds(ch2 * 16, 16)] = mv * exvs[h][j]
                    return gcarry
                lax.fori_loop(0, G, msg_body, 0)

                scatter_issue(p)
            return carry
        lax.fori_loop(0, NB // 2, pair_body, 0)
        scatter_drain(1)

        plsc.subcore_barrier()

        obase = coff + rbase
        for start, rows in CHUNKS:
            pltpu.sync_copy(num_sp.at[pl.ds(rbase + start, rows)],
                            qb0.at[pl.ds(0, rows)])
            pltpu.sync_copy(qb0.at[pl.ds(0, rows)],
                            num_out.at[pl.ds(obase + start, rows)])
            for h in range(HPC):
                pltpu.sync_copy(den_sps[h].at[pl.ds(rbase + start, rows)],
                                denT0.at[h].at[pl.ds(0, rows)])
                pltpu.sync_copy(denT0.at[h].at[pl.ds(0, rows)],
                                den_outs[h].at[pl.ds(obase + start, rows)])

        @pl.when(sub == NSUB - 1)
        def _out_tail():
            pltpu.sync_copy(num_sp.at[pl.ds(NSUB * RPS, 16)],
                            qb0.at[pl.ds(0, 16)])
            pltpu.sync_copy(qb0.at[pl.ds(0, 16)],
                            num_out.at[pl.ds(coff + NSUB * RPS, 16)])
            for h in range(HPC):
                pltpu.sync_copy(den_sps[h].at[pl.ds(NSUB * RPS, 16)],
                                denT0.at[h].at[pl.ds(0, 16)])
                pltpu.sync_copy(denT0.at[h].at[pl.ds(0, 16)],
                                den_outs[h].at[pl.ds(coff + NSUB * RPS, 16)])

    return edge_kernel


def _make_final(hc, use_gelu):
    D = hc // 2
    c = hc // H
    bn = 400
    grid = NN // bn

    def body(num_ref, den_ref, skip_ref, out_ref):
        ih = lax.broadcasted_iota(jnp.int32, (HPC, D), 0)
        ic = lax.broadcasted_iota(jnp.int32, (HPC, D), 1)
        R = (ic // c == ih).astype(jnp.float32)
        halves = []
        for half in range(2):
            dexp = jnp.dot(den_ref[half], R,
                           preferred_element_type=jnp.float32)
            halves.append(num_ref[half, :, :D] / (dexp + 1e-16))
        out = jnp.concatenate(halves, axis=1) + skip_ref[...]
        if use_gelu:
            out = 0.5 * out * (1.0 + lax.erf(out * (1.0 / math.sqrt(2.0))))
        out_ref[...] = out

    return pl.pallas_call(
        body,
        grid=(grid,),
        in_specs=[
            pl.BlockSpec((2, bn, W), lambda i: (0, i, 0)),
            pl.BlockSpec((2, bn, HPC), lambda i: (0, i, 0)),
            pl.BlockSpec((bn, hc), lambda i: (i, 0)),
        ],
        out_specs=pl.BlockSpec((bn, hc), lambda i: (i, 0)),
        out_shape=jax.ShapeDtypeStruct((NN, hc), jnp.float32),
    )


_PROJ = {}
_EDGE = {}
_FINAL = {}
for _l, (_din, _hc) in enumerate(DIMS_L):
    if (_din, _hc) not in _PROJ:
        _PROJ[(_din, _hc)] = _make_proj(_din, _hc)
    if _hc not in _EDGE:
        _EDGE[_hc] = _make_edge(_hc)
    if (_hc, _l < 3) not in _FINAL:
        _FINAL[(_hc, _l < 3)] = _make_final(_hc, _l < 3)


def kernel(x, edge_index,
           Wq0, bq0, Wk0, bk0, Wv0, bv0, Ws0, bs0,
           Wq1, bq1, Wk1, bk1, Wv1, bv1, Ws1, bs1,
           Wq2, bq2, Wk2, bk2, Wv2, bv2, Ws2, bs2,
           Wq3, bq3, Wk3, bk3, Wv3, bv3, Ws3, bs3):
    params = (Wq0, bq0, Wk0, bk0, Wv0, bv0, Ws0, bs0,
              Wq1, bq1, Wk1, bk1, Wv1, bv1, Ws1, bs1,
              Wq2, bq2, Wk2, bk2, Wv2, bv2, Ws2, bs2,
              Wq3, bq3, Wk3, bk3, Wv3, bv3, Ws3, bs3)
    srcp = jnp.pad(edge_index[0], (0, EPAD))
    dstp = jnp.pad(edge_index[1], (0, EPAD))
    h = x
    for l, (din, hc) in enumerate(DIMS_L):
        Wq, bq, Wk, bk, Wv, bv, Ws, bs = params[8 * l:8 * (l + 1)]
        Wc = jnp.concatenate([Wq, Wk, Wv, Ws], axis=1)
        bc = jnp.concatenate([bq, bk, bv, bs]).reshape(1, -1)
        q3, k3, v3, skip = _PROJ[(din, hc)](h, Wc, bc)
        num, d0, d1, d2, d3 = _EDGE[hc](
            q3.reshape(2 * NN, W), k3.reshape(2 * NN, W),
            v3.reshape(2 * NN, W), srcp, dstp)
        den = jnp.stack([d0, d1, d2, d3], axis=-1).reshape(2, NN, HPC)
        h = _FINAL[(hc, l < 3)](num.reshape(2, NN, W), den, skip)
    return h


# speedup vs baseline: 6.8223x; 1.0777x over previous
import functools
import math

import jax
import jax.numpy as jnp
from jax import lax
from jax.experimental import pallas as pl
from jax.experimental.pallas import tpu as pltpu
from jax.experimental.pallas import tpu_sc as plsc

NN = 10000
EE = 320000
H = 8
NSUB = 16
NCORE = 2
EC = EE // NSUB
RPS = 624
EPAD = 160
HPC = 4
GEOM = {256: (64, 314, 128), 128: (64, 314, 128)}
DIMS_L = [(128, 256), (256, 256), (256, 256), (256, 128)]


def _make_proj(din, hc):
    D = hc // 2
    W = GEOM[hc][2]
    bn = 400
    grid = NN // bn
    wdim = 4 * hc

    def body(x_ref, w_ref, b_ref, q_ref, k_ref, v_ref, s_ref):
        acc = jnp.dot(x_ref[...], w_ref[...],
                      preferred_element_type=jnp.float32) + b_ref[...]
        pad = jnp.zeros((bn, W - D), jnp.float32) if D < W else None
        for t, ref in enumerate([q_ref, k_ref, v_ref]):
            blk = acc[:, t * hc:(t + 1) * hc]
            for half in range(2):
                hb = blk[:, half * D:(half + 1) * D]
                if pad is not None:
                    hb = jnp.concatenate([hb, pad], axis=1)
                ref[half] = hb
        s_ref[...] = acc[:, 3 * hc:]

    return pl.pallas_call(
        body,
        grid=(grid,),
        in_specs=[
            pl.BlockSpec((bn, din), lambda i: (i, 0)),
            pl.BlockSpec((din, wdim), lambda i: (0, 0)),
            pl.BlockSpec((1, wdim), lambda i: (0, 0)),
        ],
        out_specs=[
            pl.BlockSpec((2, bn, W), lambda i: (0, i, 0)),
            pl.BlockSpec((2, bn, W), lambda i: (0, i, 0)),
            pl.BlockSpec((2, bn, W), lambda i: (0, i, 0)),
            pl.BlockSpec((bn, hc), lambda i: (i, 0)),
        ],
        out_shape=[
            jax.ShapeDtypeStruct((2, NN, W), jnp.float32),
            jax.ShapeDtypeStruct((2, NN, W), jnp.float32),
            jax.ShapeDtypeStruct((2, NN, W), jnp.float32),
            jax.ShapeDtypeStruct((NN, hc), jnp.float32),
        ],
    )


def _make_edge(hc):
    CH = hc // 2
    c = hc // H
    inv = 1.0 / math.sqrt(c)
    B, NB, W = GEOM[hc]
    G = B // 16
    CHUNKS = [(i * B, B) for i in range(RPS // B)]
    if RPS % B:
        CHUNKS.append((RPS - RPS % B, RPS % B))
    mesh = plsc.VectorSubcoreMesh(core_axis_name="c", subcore_axis_name="s")

    @functools.partial(
        pl.kernel,
        mesh=mesh,
        compiler_params=pltpu.CompilerParams(needs_layout_passes=False),
        out_type=(
            jax.ShapeDtypeStruct((NCORE * NN, W), jnp.float32),
            jax.ShapeDtypeStruct((NCORE * NN,), jnp.float32),
            jax.ShapeDtypeStruct((NCORE * NN,), jnp.float32),
            jax.ShapeDtypeStruct((NCORE * NN,), jnp.float32),
            jax.ShapeDtypeStruct((NCORE * NN,), jnp.float32),
        ),
        scratch_types=[
            pltpu.VMEM((B, W), jnp.float32),
            pltpu.VMEM((B, W), jnp.float32),
            pltpu.VMEM((B, W), jnp.float32),
            pltpu.VMEM((B, W), jnp.float32),
            pltpu.VMEM((B, W), jnp.float32),
            pltpu.VMEM((HPC, B), jnp.float32),
            pltpu.VMEM((HPC, B), jnp.float32),
            pltpu.VMEM((B,), jnp.int32),
            pltpu.VMEM((B,), jnp.int32),
            pltpu.VMEM((B,), jnp.int32),
            pltpu.VMEM((B,), jnp.int32),
            pltpu.VMEM((B,), jnp.int32),
            pltpu.VMEM((B,), jnp.int32),
            pltpu.VMEM((B,), jnp.int32),
            pltpu.VMEM((B,), jnp.int32),
            pltpu.VMEM((B,), jnp.int32),
            pltpu.VMEM((B,), jnp.int32),
            pltpu.VMEM_SHARED((NN, W), jnp.float32),
            pltpu.VMEM_SHARED((NN,), jnp.float32),
            pltpu.VMEM_SHARED((NN,), jnp.float32),
            pltpu.VMEM_SHARED((NN,), jnp.float32),
            pltpu.VMEM_SHARED((NN,), jnp.float32),
            pltpu.SemaphoreType.DMA,
            pltpu.SemaphoreType.DMA,
            pltpu.SemaphoreType.DMA,
            pltpu.SemaphoreType.DMA,
            pltpu.SemaphoreType.DMA,
        ],
    )
    def edge_kernel(q_hbm, k_hbm, v_hbm, src_hbm, dst_hbm,
                    num_out, den_out0, den_out1, den_out2, den_out3,
                    qb0, qb1, kb0, kb1, vb, denT0, denT1,
                    sraw0, sraw1, draw0, draw1,
                    srcg0, srcg1, dstg0, dstg1, dstv0, dstv1,
                    num_sp, den_sp0, den_sp1, den_sp2, den_sp3,
                    gsem0, gsem1, vsem, ssem, isem):
        qb = [qb0, qb1]
        kb = [kb0, kb1]
        denT = [denT0, denT1]
        sraw = [sraw0, sraw1]
        draw = [draw0, draw1]
        srcg = [srcg0, srcg1]
        dstg = [dstg0, dstg1]
        dstv = [dstv0, dstv1]
        gsem = [gsem0, gsem1]
        den_sps = [den_sp0, den_sp1, den_sp2, den_sp3]
        den_outs = [den_out0, den_out1, den_out2, den_out3]
        core = lax.axis_index("c")
        sub = lax.axis_index("s")
        lane = lax.iota(jnp.int32, 16)
        zero16 = jnp.zeros((16,), jnp.float32)
        coff = core * NN
        ebase = sub * EC

        def zrow(r, carry):
            for j in range(W // 16):
                qb0[r, pl.ds(j * 16, 16)] = zero16
            return carry
        lax.fori_loop(0, B, zrow, 0)
        for h in range(HPC):
            for j in range(B // 16):
                denT0[h, pl.ds(j * 16, 16)] = zero16

        rbase = sub * RPS
        for start, rows in CHUNKS:
            pltpu.sync_copy(qb0.at[pl.ds(0, rows)],
                            num_sp.at[pl.ds(rbase + start, rows)])
            for h in range(HPC):
                pltpu.sync_copy(denT0.at[h].at[pl.ds(0, rows)],
                                den_sps[h].at[pl.ds(rbase + start, rows)])

        @pl.when(sub == NSUB - 1)
        def _init_tail():
            pltpu.sync_copy(qb0.at[pl.ds(0, 16)],
                            num_sp.at[pl.ds(NSUB * RPS, 16)])
            for h in range(HPC):
                pltpu.sync_copy(denT0.at[h].at[pl.ds(0, 16)],
                                den_sps[h].at[pl.ds(NSUB * RPS, 16)])

        plsc.subcore_barrier()

        def idx_load(t, p):
            off = ebase + t * B
            a = pltpu.async_copy(src_hbm.at[pl.ds(off, B)], sraw[p], isem)
            b_ = pltpu.async_copy(dst_hbm.at[pl.ds(off, B)], draw[p], isem)
            return a, b_

        def idx_wait(p):
            pltpu.make_async_copy(src_hbm.at[pl.ds(0, B)], sraw[p],
                                  isem).wait()
            pltpu.make_async_copy(dst_hbm.at[pl.ds(0, B)], draw[p],
                                  isem).wait()

        def offset_idx(p):
            for i in range(B // 16):
                sl = pl.ds(i * 16, 16)
                srcg[p][sl] = sraw[p][sl] + coff
                dstg[p][sl] = draw[p][sl] + coff
                dstv[p][sl] = draw[p][sl]

        def gather_issue(p):
            pltpu.async_copy(q_hbm.at[dstg[p]], qb[p], gsem[p])
            pltpu.async_copy(k_hbm.at[srcg[p]], kb[p], gsem[p])

        def gather_wait(p):
            pltpu.make_async_copy(q_hbm.at[dstg[p]], qb[p], gsem[p]).wait()
            pltpu.make_async_copy(k_hbm.at[srcg[p]], kb[p], gsem[p]).wait()

        def scatter_issue(p):
            pltpu.async_copy(qb[p], num_sp.at[dstv[p]], ssem, add=True)
            for h in range(HPC):
                pltpu.async_copy(denT[p].at[h], den_sps[h].at[dstv[p]],
                                 ssem, add=True)

        def scatter_drain(p):
            pltpu.make_async_copy(qb[p], num_sp.at[dstv[p]], ssem).wait()
            for h in range(HPC):
                pltpu.make_async_copy(denT[p].at[h],
                                      den_sps[h].at[dstv[p]], ssem).wait()

        a0, b0 = idx_load(0, 0)
        a0.wait()
        b0.wait()
        offset_idx(0)
        gather_issue(0)
        idx_load(1, 1)

        def pair_body(i, carry):
            for p in (0, 1):
                t = 2 * i + p

                @pl.when(t > 0)
                def _drain_prev():
                    scatter_drain(1 - p)

                gather_wait(p)
                pltpu.async_copy(v_hbm.at[srcg[p]], vb, vsem)

                @pl.when(t + 1 < NB)
                def _stage_next():
                    idx_wait(1 - p)
                    offset_idx(1 - p)
                    gather_issue(1 - p)

                @pl.when(t + 2 < NB)
                def _fetch_next_idx():
                    idx_load(t + 2, p)

                def alpha_body(g, gcarry):
                    alph = [zero16] * HPC
                    for j in range(16):
                        erow = g * 16 + j
                        for h in range(HPC):
                            s = zero16
                            for c2 in range(c // 16):
                                sl = pl.ds(h * c + c2 * 16, 16)
                                s = s + qb[p][erow, sl] * kb[p][erow, sl]
                            a = jnp.sum(s) * inv
                            alph[h] = jnp.where(lane == j, a, alph[h])
                    live = (t * B + g * 16 + lane) < EC
                    for h in range(HPC):
                        ex = jnp.where(live, jnp.exp(alph[h]), 0.0)
                        denT[p][h, pl.ds(g * 16, 16)] = ex
                    return gcarry
                lax.fori_loop(0, G, alpha_body, 0)

                pltpu.make_async_copy(v_hbm.at[srcg[p]], vb, vsem).wait()

                def msg_body(g, gcarry):
                    exvs = [denT[p][h, pl.ds(g * 16, 16)]
                            for h in range(HPC)]
                    for j in range(16):
                        erow = g * 16 + j
                        for ch2 in range(CH // 16):
                            h = (ch2 * 16) // c
                            mv = vb[erow, pl.ds(ch2 * 16, 16)]
                            qb[p][erow, pl.ds(ch2 * 16, 16)] = mv * exvs[h][j]
                    return gcarry
                lax.fori_loop(0, G, msg_body, 0)

                scatter_issue(p)
            return carry
        lax.fori_loop(0, NB // 2, pair_body, 0)
        scatter_drain(1)

        plsc.subcore_barrier()

        obase = coff + rbase
        for start, rows in CHUNKS:
            pltpu.sync_copy(num_sp.at[pl.ds(rbase + start, rows)],
                            qb0.at[pl.ds(0, rows)])
            pltpu.sync_copy(qb0.at[pl.ds(0, rows)],
                            num_out.at[pl.ds(obase + start, rows)])
            for h in range(HPC):
                pltpu.sync_copy(den_sps[h].at[pl.ds(rbase + start, rows)],
                                denT0.at[h].at[pl.ds(0, rows)])
                pltpu.sync_copy(denT0.at[h].at[pl.ds(0, rows)],
                                den_outs[h].at[pl.ds(obase + start, rows)])

        @pl.when(sub == NSUB - 1)
        def _out_tail():
            pltpu.sync_copy(num_sp.at[pl.ds(NSUB * RPS, 16)],
                            qb0.at[pl.ds(0, 16)])
            pltpu.sync_copy(qb0.at[pl.ds(0, 16)],
                            num_out.at[pl.ds(coff + NSUB * RPS, 16)])
            for h in range(HPC):
                pltpu.sync_copy(den_sps[h].at[pl.ds(NSUB * RPS, 16)],
                                denT0.at[h].at[pl.ds(0, 16)])
                pltpu.sync_copy(denT0.at[h].at[pl.ds(0, 16)],
                                den_outs[h].at[pl.ds(coff + NSUB * RPS, 16)])

    return edge_kernel


def _make_final(hc, use_gelu):
    D = hc // 2
    c = hc // H
    W = GEOM[hc][2]
    bn = 400
    grid = NN // bn

    def body(num_ref, den_ref, skip_ref, out_ref):
        ih = lax.broadcasted_iota(jnp.int32, (HPC, D), 0)
        ic = lax.broadcasted_iota(jnp.int32, (HPC, D), 1)
        R = (ic // c == ih).astype(jnp.float32)
        halves = []
        for half in range(2):
            dexp = jnp.dot(den_ref[half], R,
                           preferred_element_type=jnp.float32)
            halves.append(num_ref[half, :, :D] / (dexp + 1e-16))
        out = jnp.concatenate(halves, axis=1) + skip_ref[...]
        if use_gelu:
            out = 0.5 * out * (1.0 + lax.erf(out * (1.0 / math.sqrt(2.0))))
        out_ref[...] = out

    return pl.pallas_call(
        body,
        grid=(grid,),
        in_specs=[
            pl.BlockSpec((2, bn, W), lambda i: (0, i, 0)),
            pl.BlockSpec((2, bn, HPC), lambda i: (0, i, 0)),
            pl.BlockSpec((bn, hc), lambda i: (i, 0)),
        ],
        out_specs=pl.BlockSpec((bn, hc), lambda i: (i, 0)),
        out_shape=jax.ShapeDtypeStruct((NN, hc), jnp.float32),
    )


_PROJ = {}
_EDGE = {}
_FINAL = {}
for _l, (_din, _hc) in enumerate(DIMS_L):
    if (_din, _hc) not in _PROJ:
        _PROJ[(_din, _hc)] = _make_proj(_din, _hc)
    if _hc not in _EDGE:
        _EDGE[_hc] = _make_edge(_hc)
    if (_hc, _l < 3) not in _FINAL:
        _FINAL[(_hc, _l < 3)] = _make_final(_hc, _l < 3)


def kernel(x, edge_index,
           Wq0, bq0, Wk0, bk0, Wv0, bv0, Ws0, bs0,
           Wq1, bq1, Wk1, bk1, Wv1, bv1, Ws1, bs1,
           Wq2, bq2, Wk2, bk2, Wv2, bv2, Ws2, bs2,
           Wq3, bq3, Wk3, bk3, Wv3, bv3, Ws3, bs3):
    params = (Wq0, bq0, Wk0, bk0, Wv0, bv0, Ws0, bs0,
              Wq1, bq1, Wk1, bk1, Wv1, bv1, Ws1, bs1,
              Wq2, bq2, Wk2, bk2, Wv2, bv2, Ws2, bs2,
              Wq3, bq3, Wk3, bk3, Wv3, bv3, Ws3, bs3)
    srcp = jnp.pad(edge_index[0], (0, EPAD))
    dstp = jnp.pad(edge_index[1], (0, EPAD))
    h = x
    for l, (din, hc) in enumerate(DIMS_L):
        Wq, bq, Wk, bk, Wv, bv, Ws, bs = params[8 * l:8 * (l + 1)]
        Wc = jnp.concatenate([Wq, Wk, Wv, Ws], axis=1)
        bc = jnp.concatenate([bq, bk, bv, bs]).reshape(1, -1)
        WL = GEOM[hc][2]
        q3, k3, v3, skip = _PROJ[(din, hc)](h, Wc, bc)
        num, d0, d1, d2, d3 = _EDGE[hc](
            q3.reshape(2 * NN, WL), k3.reshape(2 * NN, WL),
            v3.reshape(2 * NN, WL), srcp, dstp)
        den = jnp.stack([d0, d1, d2, d3], axis=-1).reshape(2, NN, HPC)
        h = _FINAL[(hc, l < 3)](num.reshape(2, NN, WL), den, skip)
    return h


# speedup vs baseline: 6.9328x; 1.0162x over previous
import functools
import math

import jax
import jax.numpy as jnp
from jax import lax
from jax.experimental import pallas as pl
from jax.experimental.pallas import tpu as pltpu
from jax.experimental.pallas import tpu_sc as plsc

NN = 10000
EE = 320000
H = 8
NSUB = 16
NCORE = 2
EC = EE // NSUB
RPS = 624
EPAD = 160
HPC = 4
GEOM = {256: (64, 314, 128), 128: (64, 314, 128)}
DIMS_L = [(128, 256), (256, 256), (256, 256), (256, 128)]


def _make_proj(din, hc):
    D = hc // 2
    W = GEOM[hc][2]
    bn = 400
    grid = NN // bn
    wdim = 4 * hc

    def body(x_ref, w_ref, b_ref, q_ref, k_ref, v_ref, s_ref):
        acc = jnp.dot(x_ref[...], w_ref[...],
                      preferred_element_type=jnp.float32) + b_ref[...]
        pad = jnp.zeros((bn, W - D), jnp.float32) if D < W else None
        for t, ref in enumerate([q_ref, k_ref, v_ref]):
            blk = acc[:, t * hc:(t + 1) * hc]
            for half in range(2):
                hb = blk[:, half * D:(half + 1) * D]
                if pad is not None:
                    hb = jnp.concatenate([hb, pad], axis=1)
                ref[half] = hb
        s_ref[...] = acc[:, 3 * hc:]

    return pl.pallas_call(
        body,
        grid=(grid,),
        in_specs=[
            pl.BlockSpec((bn, din), lambda i: (i, 0)),
            pl.BlockSpec((din, wdim), lambda i: (0, 0)),
            pl.BlockSpec((1, wdim), lambda i: (0, 0)),
        ],
        out_specs=[
            pl.BlockSpec((2, bn, W), lambda i: (0, i, 0)),
            pl.BlockSpec((2, bn, W), lambda i: (0, i, 0)),
            pl.BlockSpec((2, bn, W), lambda i: (0, i, 0)),
            pl.BlockSpec((bn, hc), lambda i: (i, 0)),
        ],
        out_shape=[
            jax.ShapeDtypeStruct((2, NN, W), jnp.float32),
            jax.ShapeDtypeStruct((2, NN, W), jnp.float32),
            jax.ShapeDtypeStruct((2, NN, W), jnp.float32),
            jax.ShapeDtypeStruct((NN, hc), jnp.float32),
        ],
    )


def _make_edge(hc):
    CH = hc // 2
    c = hc // H
    inv = 1.0 / math.sqrt(c)
    B, NB, W = GEOM[hc]
    G = B // 16
    CHUNKS = [(i * B, B) for i in range(RPS // B)]
    if RPS % B:
        CHUNKS.append((RPS - RPS % B, RPS % B))
    mesh = plsc.VectorSubcoreMesh(core_axis_name="c", subcore_axis_name="s")

    @functools.partial(
        pl.kernel,
        mesh=mesh,
        compiler_params=pltpu.CompilerParams(needs_layout_passes=False),
        out_type=(
            jax.ShapeDtypeStruct((NCORE * NN, W), jnp.float32),
            jax.ShapeDtypeStruct((NCORE * NN,), jnp.float32),
            jax.ShapeDtypeStruct((NCORE * NN,), jnp.float32),
            jax.ShapeDtypeStruct((NCORE * NN,), jnp.float32),
            jax.ShapeDtypeStruct((NCORE * NN,), jnp.float32),
        ),
        scratch_types=[
            pltpu.VMEM((B, W), jnp.float32),
            pltpu.VMEM((B, W), jnp.float32),
            pltpu.VMEM((B, W), jnp.float32),
            pltpu.VMEM((B, W), jnp.float32),
            pltpu.VMEM((B, W), jnp.float32),
            pltpu.VMEM((HPC, B), jnp.float32),
            pltpu.VMEM((HPC, B), jnp.float32),
            pltpu.VMEM((B,), jnp.int32),
            pltpu.VMEM((B,), jnp.int32),
            pltpu.VMEM((B,), jnp.int32),
            pltpu.VMEM((B,), jnp.int32),
            pltpu.VMEM((B,), jnp.int32),
            pltpu.VMEM((B,), jnp.int32),
            pltpu.VMEM((B,), jnp.int32),
            pltpu.VMEM((B,), jnp.int32),
            pltpu.VMEM((B,), jnp.int32),
            pltpu.VMEM((B,), jnp.int32),
            pltpu.VMEM_SHARED((NN, W), jnp.float32),
            pltpu.VMEM_SHARED((NN,), jnp.float32),
            pltpu.VMEM_SHARED((NN,), jnp.float32),
            pltpu.VMEM_SHARED((NN,), jnp.float32),
            pltpu.VMEM_SHARED((NN,), jnp.float32),
            pltpu.SemaphoreType.DMA,
            pltpu.SemaphoreType.DMA,
            pltpu.SemaphoreType.DMA,
            pltpu.SemaphoreType.DMA,
            pltpu.SemaphoreType.DMA,
        ],
    )
    def edge_kernel(q_hbm, k_hbm, v_hbm, src_hbm, dst_hbm,
                    num_out, den_out0, den_out1, den_out2, den_out3,
                    qb0, qb1, kb0, kb1, vb, denT0, denT1,
                    sraw0, sraw1, draw0, draw1,
                    srcg0, srcg1, dstg0, dstg1, dstv0, dstv1,
                    num_sp, den_sp0, den_sp1, den_sp2, den_sp3,
                    gsem0, gsem1, vsem, ssem, isem):
        qb = [qb0, qb1]
        kb = [kb0, kb1]
        denT = [denT0, denT1]
        sraw = [sraw0, sraw1]
        draw = [draw0, draw1]
        srcg = [srcg0, srcg1]
        dstg = [dstg0, dstg1]
        dstv = [dstv0, dstv1]
        gsem = [gsem0, gsem1]
        den_sps = [den_sp0, den_sp1, den_sp2, den_sp3]
        den_outs = [den_out0, den_out1, den_out2, den_out3]
        core = lax.axis_index("c")
        sub = lax.axis_index("s")
        lane = lax.iota(jnp.int32, 16)
        zero16 = jnp.zeros((16,), jnp.float32)
        coff = core * NN
        ebase = sub * EC

        def zrow(r, carry):
            for j in range(W // 16):
                qb0[r, pl.ds(j * 16, 16)] = zero16
            return carry
        lax.fori_loop(0, B, zrow, 0)
        for h in range(HPC):
            for j in range(B // 16):
                denT0[h, pl.ds(j * 16, 16)] = zero16

        rbase = sub * RPS
        for start, rows in CHUNKS:
            pltpu.sync_copy(qb0.at[pl.ds(0, rows)],
                            num_sp.at[pl.ds(rbase + start, rows)])
            for h in range(HPC):
                pltpu.sync_copy(denT0.at[h].at[pl.ds(0, rows)],
                                den_sps[h].at[pl.ds(rbase + start, rows)])

        @pl.when(sub == NSUB - 1)
        def _init_tail():
            pltpu.sync_copy(qb0.at[pl.ds(0, 16)],
                            num_sp.at[pl.ds(NSUB * RPS, 16)])
            for h in range(HPC):
                pltpu.sync_copy(denT0.at[h].at[pl.ds(0, 16)],
                                den_sps[h].at[pl.ds(NSUB * RPS, 16)])

        plsc.subcore_barrier()

        def idx_load(t, p):
            off = ebase + t * B
            a = pltpu.async_copy(src_hbm.at[pl.ds(off, B)], sraw[p], isem)
            b_ = pltpu.async_copy(dst_hbm.at[pl.ds(off, B)], draw[p], isem)
            return a, b_

        def idx_wait(p):
            pltpu.make_async_copy(src_hbm.at[pl.ds(0, B)], sraw[p],
                                  isem).wait()
            pltpu.make_async_copy(dst_hbm.at[pl.ds(0, B)], draw[p],
                                  isem).wait()

        def offset_idx(p):
            for i in range(B // 16):
                sl = pl.ds(i * 16, 16)
                srcg[p][sl] = sraw[p][sl] + coff
                dstg[p][sl] = draw[p][sl] + coff
                dstv[p][sl] = draw[p][sl]

        def gather_issue(p):
            pltpu.async_copy(q_hbm.at[dstg[p]], qb[p], gsem[p])
            pltpu.async_copy(k_hbm.at[srcg[p]], kb[p], gsem[p])

        def gather_wait(p):
            pltpu.make_async_copy(q_hbm.at[dstg[p]], qb[p], gsem[p]).wait()
            pltpu.make_async_copy(k_hbm.at[srcg[p]], kb[p], gsem[p]).wait()

        def scatter_issue(p):
            pltpu.async_copy(qb[p], num_sp.at[dstv[p]], ssem, add=True)
            for h in range(HPC):
                pltpu.async_copy(denT[p].at[h], den_sps[h].at[dstv[p]],
                                 ssem, add=True)

        def scatter_drain(p):
            pltpu.make_async_copy(qb[p], num_sp.at[dstv[p]], ssem).wait()
            for h in range(HPC):
                pltpu.make_async_copy(denT[p].at[h],
                                      den_sps[h].at[dstv[p]], ssem).wait()

        a0, b0 = idx_load(0, 0)
        a0.wait()
        b0.wait()
        offset_idx(0)
        gather_issue(0)
        idx_load(1, 1)

        def pair_body(i, carry):
            for p in (0, 1):
                t = 2 * i + p

                @pl.when(t > 0)
                def _drain_prev():
                    scatter_drain(1 - p)

                gather_wait(p)
                pltpu.async_copy(v_hbm.at[srcg[p]], vb, vsem)

                @pl.when(t + 1 < NB)
                def _stage_next():
                    idx_wait(1 - p)
                    offset_idx(1 - p)
                    gather_issue(1 - p)

                @pl.when(t + 2 < NB)
                def _fetch_next_idx():
                    idx_load(t + 2, p)

                def alpha_body(g, gcarry):
                    alph = [zero16] * HPC
                    for j in range(16):
                        erow = g * 16 + j
                        for h in range(HPC):
                            s = zero16
                            for c2 in range(c // 16):
                                sl = pl.ds(h * c + c2 * 16, 16)
                                s = s + qb[p][erow, sl] * kb[p][erow, sl]
                            a = jnp.sum(s) * inv
                            alph[h] = jnp.where(lane == j, a, alph[h])
                    live = (t * B + g * 16 + lane) < EC
                    for h in range(HPC):
                        ex = jnp.where(live, jnp.exp(alph[h]), 0.0)
                        denT[p][h, pl.ds(g * 16, 16)] = ex
                    return gcarry
                lax.fori_loop(0, G, alpha_body, 0)

                pltpu.make_async_copy(v_hbm.at[srcg[p]], vb, vsem).wait()

                def msg_body(g, gcarry):
                    exvs = [denT[p][h, pl.ds(g * 16, 16)]
                            for h in range(HPC)]
                    for j in range(16):
                        erow = g * 16 + j
                        for ch2 in range(CH // 16):
                            h = (ch2 * 16) // c
                            mv = vb[erow, pl.ds(ch2 * 16, 16)]
                            qb[p][erow, pl.ds(ch2 * 16, 16)] = mv * exvs[h][j]
                    return gcarry
                lax.fori_loop(0, G, msg_body, 0)

                scatter_issue(p)
            return carry
        lax.fori_loop(0, NB // 2, pair_body, 0)
        scatter_drain(1)

        plsc.subcore_barrier()

        obase = coff + rbase
        for start, rows in CHUNKS:
            pltpu.sync_copy(num_sp.at[pl.ds(rbase + start, rows)],
                            qb0.at[pl.ds(0, rows)])
            pltpu.sync_copy(qb0.at[pl.ds(0, rows)],
                            num_out.at[pl.ds(obase + start, rows)])
            for h in range(HPC):
                pltpu.sync_copy(den_sps[h].at[pl.ds(rbase + start, rows)],
                                denT0.at[h].at[pl.ds(0, rows)])
                pltpu.sync_copy(denT0.at[h].at[pl.ds(0, rows)],
                                den_outs[h].at[pl.ds(obase + start, rows)])

        @pl.when(sub == NSUB - 1)
        def _out_tail():
            pltpu.sync_copy(num_sp.at[pl.ds(NSUB * RPS, 16)],
                            qb0.at[pl.ds(0, 16)])
            pltpu.sync_copy(qb0.at[pl.ds(0, 16)],
                            num_out.at[pl.ds(coff + NSUB * RPS, 16)])
            for h in range(HPC):
                pltpu.sync_copy(den_sps[h].at[pl.ds(NSUB * RPS, 16)],
                                denT0.at[h].at[pl.ds(0, 16)])
                pltpu.sync_copy(denT0.at[h].at[pl.ds(0, 16)],
                                den_outs[h].at[pl.ds(coff + NSUB * RPS, 16)])

    return edge_kernel


def _make_fused(din, hc):
    Dp = din // 2
    cp = din // H
    Wp = 128
    D = hc // 2
    W = GEOM[hc][2]
    bn = 400
    grid = NN // bn
    wdim = 4 * hc

    def body(num_ref, den_ref, skip_ref, w_ref, b_ref,
             q_ref, k_ref, v_ref, s_ref):
        ih = lax.broadcasted_iota(jnp.int32, (HPC, Dp), 0)
        ic = lax.broadcasted_iota(jnp.int32, (HPC, Dp), 1)
        R = (ic // cp == ih).astype(jnp.float32)
        halves = []
        for half in range(2):
            dexp = jnp.dot(den_ref[half], R,
                           preferred_element_type=jnp.float32)
            halves.append(num_ref[half, :, :Dp] / (dexp + 1e-16))
        hprev = jnp.concatenate(halves, axis=1) + skip_ref[...]
        hprev = 0.5 * hprev * (1.0 + lax.erf(hprev * (1.0 / math.sqrt(2.0))))
        acc = jnp.dot(hprev, w_ref[...],
                      preferred_element_type=jnp.float32) + b_ref[...]
        pad = jnp.zeros((bn, W - D), jnp.float32) if D < W else None
        for t, ref in enumerate([q_ref, k_ref, v_ref]):
            blk = acc[:, t * hc:(t + 1) * hc]
            for half in range(2):
                hb = blk[:, half * D:(half + 1) * D]
                if pad is not None:
                    hb = jnp.concatenate([hb, pad], axis=1)
                ref[half] = hb
        s_ref[...] = acc[:, 3 * hc:]

    return pl.pallas_call(
        body,
        grid=(grid,),
        in_specs=[
            pl.BlockSpec((2, bn, Wp), lambda i: (0, i, 0)),
            pl.BlockSpec((2, bn, HPC), lambda i: (0, i, 0)),
            pl.BlockSpec((bn, din), lambda i: (i, 0)),
            pl.BlockSpec((din, wdim), lambda i: (0, 0)),
            pl.BlockSpec((1, wdim), lambda i: (0, 0)),
        ],
        out_specs=[
            pl.BlockSpec((2, bn, W), lambda i: (0, i, 0)),
            pl.BlockSpec((2, bn, W), lambda i: (0, i, 0)),
            pl.BlockSpec((2, bn, W), lambda i: (0, i, 0)),
            pl.BlockSpec((bn, hc), lambda i: (i, 0)),
        ],
        out_shape=[
            jax.ShapeDtypeStruct((2, NN, W), jnp.float32),
            jax.ShapeDtypeStruct((2, NN, W), jnp.float32),
            jax.ShapeDtypeStruct((2, NN, W), jnp.float32),
            jax.ShapeDtypeStruct((NN, hc), jnp.float32),
        ],
    )


def _make_final(hc, use_gelu):
    D = hc // 2
    c = hc // H
    W = GEOM[hc][2]
    bn = 400
    grid = NN // bn

    def body(num_ref, den_ref, skip_ref, out_ref):
        ih = lax.broadcasted_iota(jnp.int32, (HPC, D), 0)
        ic = lax.broadcasted_iota(jnp.int32, (HPC, D), 1)
        R = (ic // c == ih).astype(jnp.float32)
        halves = []
        for half in range(2):
            dexp = jnp.dot(den_ref[half], R,
                           preferred_element_type=jnp.float32)
            halves.append(num_ref[half, :, :D] / (dexp + 1e-16))
        out = jnp.concatenate(halves, axis=1) + skip_ref[...]
        if use_gelu:
            out = 0.5 * out * (1.0 + lax.erf(out * (1.0 / math.sqrt(2.0))))
        out_ref[...] = out

    return pl.pallas_call(
        body,
        grid=(grid,),
        in_specs=[
            pl.BlockSpec((2, bn, W), lambda i: (0, i, 0)),
            pl.BlockSpec((2, bn, HPC), lambda i: (0, i, 0)),
            pl.BlockSpec((bn, hc), lambda i: (i, 0)),
        ],
        out_specs=pl.BlockSpec((bn, hc), lambda i: (i, 0)),
        out_shape=jax.ShapeDtypeStruct((NN, hc), jnp.float32),
    )


_PROJ0 = _make_proj(*DIMS_L[0])
_EDGE = {}
_FUSED = {}
for _l, (_din, _hc) in enumerate(DIMS_L):
    if _hc not in _EDGE:
        _EDGE[_hc] = _make_edge(_hc)
    if _l > 0 and (_din, _hc) not in _FUSED:
        _FUSED[(_din, _hc)] = _make_fused(_din, _hc)
_FINAL3 = _make_final(DIMS_L[3][1], False)


def kernel(x, edge_index,
           Wq0, bq0, Wk0, bk0, Wv0, bv0, Ws0, bs0,
           Wq1, bq1, Wk1, bk1, Wv1, bv1, Ws1, bs1,
           Wq2, bq2, Wk2, bk2, Wv2, bv2, Ws2, bs2,
           Wq3, bq3, Wk3, bk3, Wv3, bv3, Ws3, bs3):
    params = (Wq0, bq0, Wk0, bk0, Wv0, bv0, Ws0, bs0,
              Wq1, bq1, Wk1, bk1, Wv1, bv1, Ws1, bs1,
              Wq2, bq2, Wk2, bk2, Wv2, bv2, Ws2, bs2,
              Wq3, bq3, Wk3, bk3, Wv3, bv3, Ws3, bs3)
    srcp = jnp.pad(edge_index[0], (0, EPAD))
    dstp = jnp.pad(edge_index[1], (0, EPAD))
    num = den = skip = None
    for l, (din, hc) in enumerate(DIMS_L):
        Wq, bq, Wk, bk, Wv, bv, Ws, bs = params[8 * l:8 * (l + 1)]
        Wc = jnp.concatenate([Wq, Wk, Wv, Ws], axis=1)
        bc = jnp.concatenate([bq, bk, bv, bs]).reshape(1, -1)
        WL = GEOM[hc][2]
        if l == 0:
            q3, k3, v3, skip = _PROJ0(x, Wc, bc)
        else:
            q3, k3, v3, skip = _FUSED[(din, hc)](num, den, skip, Wc, bc)
        num, d0, d1, d2, d3 = _EDGE[hc](
            q3.reshape(2 * NN, WL), k3.reshape(2 * NN, WL),
            v3.reshape(2 * NN, WL), srcp, dstp)
        den = jnp.stack([d0, d1, d2, d3], axis=-1).reshape(2, NN, HPC)
        num = num.reshape(2, NN, WL)
    return _FINAL3(num, den, skip)
